# baseline (speedup 1.0000x reference)
import jax
import jax.numpy as jnp
from jax import lax
from jax.experimental import pallas as pl
from jax.experimental.pallas import tpu as pltpu
from jax.experimental.pallas import tpu_sc as plsc

KTOP = 64
N = 32768
ROWS = 128
LANES = 16
NV = N // LANES
GB = 64
NG = NV // GB
NCH = 2
CL = 32
NC, NS = 2, 16
NW = NC * NS
RPW = ROWS // NW
UNROLL = 16

_NEG_INF = float("-inf")
_BIG_I = 2**30


def _sc_body(x_hbm, v_hbm, i_hbm, xb, vbuf, ibuf, cref, outv, outi, sem):
    wid = lax.axis_index("s") * NC + lax.axis_index("c")
    lane = lax.iota(jnp.int32, LANES)
    zero16i = jnp.zeros((LANES,), jnp.int32)
    neginf16 = jnp.full((LANES,), _NEG_INF, jnp.float32)
    zero16f = jnp.zeros((LANES,), jnp.float32)
    bigi16 = jnp.full((LANES,), _BIG_I, jnp.int32)

    row0 = wid * RPW
    pltpu.async_copy(x_hbm.at[row0], xb.at[pl.ds(0, N)], sem)

    def do_row(r, _):
        row = row0 + r
        base = (r & 1) * N
        def init_body(j, _):
            vbuf[j] = neginf16
            return 0
        lax.fori_loop(0, NCH * CL, init_body, 0)

        pltpu.make_async_copy(
            x_hbm.at[row], xb.at[pl.ds(base, N)], sem).wait()

        @pl.when(r + 1 < RPW)
        def _():
            nbase = ((r + 1) & 1) * N
            pltpu.async_copy(
                x_hbm.at[row + 1], xb.at[pl.ds(nbase, N)], sem)

        def collect(tvec):
            def col_body(i0, curs):
                curs = list(curs)
                bi = i0 * UNROLL
                vs = [xb[pl.ds(base + (bi + u) * LANES, LANES)]
                      for u in range(UNROLL)]
                for u in range(UNROLL):
                    ch = u & (NCH - 1)
                    v = vs[u]
                    msk = v >= tvec
                    idx = lane + (bi + u) * LANES
                    addr = (curs[ch] & (CL - 1)) + ch * CL
                    plsc.store_scatter(ibuf, [addr, lane], idx, mask=msk)
                    curs[ch] = curs[ch] + msk.astype(jnp.int32)
                return tuple(curs)
            return lax.fori_loop(0, NV // UNROLL, col_body, (zero16i,) * NCH)

        tspec = jnp.full((LANES,), 2.7, jnp.float32)
        curs = collect(tspec)
        csum = curs[0]
        cmax = curs[0]
        for ch in range(1, NCH):
            csum = csum + curs[ch]
            cmax = jnp.maximum(cmax, curs[ch])
        ok = (jnp.sum(csum) >= KTOP) & (jnp.max(cmax) <= CL)
        for ch in range(NCH):
            cref[ch] = curs[ch]

        @pl.when(jnp.logical_not(ok))
        def _():
            def g_body(g, tops):
                t0, t1, t2, t3 = tops
                m = xb[pl.ds(base + g * (GB * LANES), LANES)]
                for j in range(1, GB):
                    m = jnp.maximum(
                        m, xb[pl.ds(base + g * (GB * LANES) + j * LANES, LANES)])
                hi = jnp.maximum(t0, m)
                m = jnp.minimum(t0, m)
                t0 = hi
                hi = jnp.maximum(t1, m)
                m = jnp.minimum(t1, m)
                t1 = hi
                hi = jnp.maximum(t2, m)
                m = jnp.minimum(t2, m)
                t2 = hi
                t3 = jnp.maximum(t3, m)
                return t0, t1, t2, t3

            _, _, _, t3 = lax.fori_loop(
                0, NG, g_body, (neginf16, neginf16, neginf16, neginf16))
            tvec = jnp.zeros((LANES,), jnp.float32) + jnp.min(t3)
            curs2 = collect(tvec)
            for ch in range(NCH):
                cref[ch] = curs2[ch]

        curs = tuple(cref[ch] for ch in range(NCH))

        for ch in range(NCH):
            cur_c = curs[ch]
            lmax_c = jnp.minimum(jnp.max(cur_c), CL)

            def mat_body(j, _, ch=ch, cur_c=cur_c):
                idx = ibuf[ch * CL + j]
                ok = cur_c > j
                v = plsc.load_gather(xb, [base + idx], mask=ok)
                vbuf[ch * CL + j] = jnp.where(ok, v, _NEG_INF)
                return 0

            lax.fori_loop(0, lmax_c, mat_body, 0)

            def sweep(_, carry, ch=ch, lmax_c=lmax_c):
                def ce(j, c):
                    a = ch * CL + j
                    va = vbuf[a]
                    vb = vbuf[a + 1]
                    ia = ibuf[a]
                    ib = ibuf[a + 1]
                    sw = vb > va
                    vbuf[a] = jnp.where(sw, vb, va)
                    vbuf[a + 1] = jnp.where(sw, va, vb)
                    ibuf[a] = jnp.where(sw, ib, ia)
                    ibuf[a + 1] = jnp.where(sw, ia, ib)
                    return c
                return lax.fori_loop(0, lmax_c - 1, ce, carry)

            lax.fori_loop(0, lmax_c, sweep, 0)

        lane0 = lane == 0
        def ext_body(t, st):
            ptrs = list(st)
            hvs, his = [], []
            for ch in range(NCH):
                inb = ptrs[ch] < CL
                hv = plsc.load_gather(vbuf, [ptrs[ch] + ch * CL, lane],
                                      mask=inb)
                hi = plsc.load_gather(ibuf, [ptrs[ch] + ch * CL, lane],
                                      mask=inb)
                hvs.append(jnp.where(inb, hv, _NEG_INF))
                his.append(hi)
            hvm = hvs[0]
            for ch in range(1, NCH):
                hvm = jnp.maximum(hvm, hvs[ch])
            rbest = jnp.max(hvm)
            mm = jnp.where(hvs[0] == rbest, his[0], bigi16)
            for ch in range(1, NCH):
                mm = jnp.minimum(mm, jnp.where(hvs[ch] == rbest, his[ch], bigi16))
            ibest = jnp.min(mm)
            lwin = ibest & (LANES - 1)
            selw = lane == lwin
            for ch in range(NCH):
                upd = selw & (hvs[ch] == rbest) & (his[ch] == ibest)
                ptrs[ch] = ptrs[ch] + upd.astype(jnp.int32)

            tvecidx = zero16i + t
            plsc.store_scatter(outv, [tvecidx], zero16f + rbest, mask=lane0)
            plsc.store_scatter(outi, [tvecidx], zero16i + ibest, mask=lane0)
            return tuple(ptrs)

        st0 = (zero16i,) * NCH
        lax.fori_loop(0, KTOP, ext_body, st0)

        pltpu.sync_copy(outv, v_hbm.at[row])
        pltpu.sync_copy(outi, i_hbm.at[row])
        return 0

    lax.fori_loop(0, RPW, do_row, 0)


def kernel(x):
    mesh = plsc.VectorSubcoreMesh(
        core_axis_name="c", subcore_axis_name="s", num_cores=NC, num_subcores=NS)
    f = pl.kernel(
        _sc_body,
        out_type=(
            jax.ShapeDtypeStruct((ROWS, KTOP), jnp.float32),
            jax.ShapeDtypeStruct((ROWS, KTOP), jnp.int32),
        ),
        mesh=mesh,
        compiler_params=pltpu.CompilerParams(needs_layout_passes=False, use_tc_tiling_on_sc=True),
        scratch_types=[
            pltpu.VMEM((2 * N,), jnp.float32),
            pltpu.VMEM((NCH * CL, LANES), jnp.float32),
            pltpu.VMEM((NCH * CL, LANES), jnp.int32),
            pltpu.VMEM((NCH, LANES), jnp.int32),
            pltpu.VMEM((KTOP,), jnp.float32),
            pltpu.VMEM((KTOP,), jnp.int32),
            pltpu.SemaphoreType.DMA,
        ],
    )
    return f(x)
